# baseline (speedup 1.0000x reference)
import functools

import jax
import jax.numpy as jnp
from jax import lax
from jax.experimental import pallas as pl
from jax.experimental.pallas import tpu as pltpu
from jax.experimental.pallas import tpu_sc as plsc

_B = 16384
_D = 16
_NC = 2
_NS = 16
_NW = _NC * _NS
_BPW = _B // _NW
_CHUNK = 128
_NCH = _BPW // _CHUNK
_L = 16


def _extract(rows_v, idx_ref, j, stage):
    for g in range(_CHUNK // _L):
        pos = lax.iota(jnp.int32, _L) + (g * _L)
        ids = idx_ref[j, 0, pl.ds(g * _L, _L)]
        sub = (ids & 7) * _D
        for k in range(_D):
            vals = plsc.load_gather(rows_v, [pos, sub + k])
            stage[k // 8, k % 8, pl.ds(g * _L, _L)] = vals


def _embed_body(user_hbm, item_hbm, uw_hbm, iw_hbm, out_u, out_i,
                idx_u, idx_i, rows_u, rows_i, stage_u, stage_i, sem):
    wid = lax.axis_index("s") * _NC + lax.axis_index("c")
    pltpu.sync_copy(user_hbm.at[wid], idx_u.at[pl.ds(0, _NCH)])
    pltpu.sync_copy(item_hbm.at[wid], idx_i.at[pl.ds(0, _NCH)])
    for j in range(_NCH):
        for g in range(_CHUNK // _L):
            sl = pl.ds(g * _L, _L)
            idx_u[_NCH + j, 0, sl] = idx_u[j, 0, sl] >> 3
            idx_i[_NCH + j, 0, sl] = idx_i[j, 0, sl] >> 3

    cps_u = [None] * _NCH
    cps_i = [None] * _NCH

    def fire_u(j):
        cps_u[j] = pltpu.async_copy(
            uw_hbm.at[idx_u.at[_NCH + j, 0]], rows_u.at[j % 2], sem)

    def fire_i(j):
        cps_i[j] = pltpu.async_copy(
            iw_hbm.at[idx_i.at[_NCH + j, 0]], rows_i.at[j % 2], sem)

    fire_u(0)
    fire_i(0)
    fire_u(1)
    fire_i(1)
    for j in range(_NCH):
        c = wid * _NCH + j
        cps_u[j].wait()
        _extract(rows_u.at[j % 2], idx_u, j, stage_u)
        pltpu.sync_copy(stage_u, out_u.at[:, c])
        if j + 2 < _NCH:
            fire_u(j + 2)
        cps_i[j].wait()
        _extract(rows_i.at[j % 2], idx_i, j, stage_i)
        pltpu.sync_copy(stage_i, out_i.at[:, c])
        if j + 2 < _NCH:
            fire_i(j + 2)


@jax.jit
def kernel(user, item, embed_user_w, embed_item_w):
    call = functools.partial(
        pl.kernel,
        mesh=plsc.VectorSubcoreMesh(core_axis_name="c", subcore_axis_name="s"),
        compiler_params=pltpu.CompilerParams(
            use_tc_tiling_on_sc=True, needs_layout_passes=False),
        out_type=(
            jax.ShapeDtypeStruct((2, _B // _CHUNK, 8, _CHUNK), jnp.float32),
            jax.ShapeDtypeStruct((2, _B // _CHUNK, 8, _CHUNK), jnp.float32),
        ),
        scratch_types=[
            pltpu.VMEM((2 * _NCH, 1, _CHUNK), jnp.int32),
            pltpu.VMEM((2 * _NCH, 1, _CHUNK), jnp.int32),
            pltpu.VMEM((2, _CHUNK, 128), jnp.float32),
            pltpu.VMEM((2, _CHUNK, 128), jnp.float32),
            pltpu.VMEM((2, 8, _CHUNK), jnp.float32),
            pltpu.VMEM((2, 8, _CHUNK), jnp.float32),
            pltpu.SemaphoreType.DMA,
        ],
    )(_embed_body)
    uw = embed_user_w.reshape(-1, 128)
    iw = jnp.pad(embed_item_w, ((0, 7), (0, 0))).reshape(-1, 128)
    u4 = user.reshape(_NW, _NCH, 1, _CHUNK)
    i4 = item.reshape(_NW, _NCH, 1, _CHUNK)
    out_u, out_i = call(u4, i4, uw, iw)
    ou = out_u.transpose(1, 3, 0, 2).reshape(_B, _D)
    oi = out_i.transpose(1, 3, 0, 2).reshape(_B, _D)
    return ou, oi


# speedup vs baseline: 1.3767x; 1.3767x over previous
import functools

import jax
import jax.numpy as jnp
from jax import lax
from jax.experimental import pallas as pl
from jax.experimental.pallas import tpu as pltpu
from jax.experimental.pallas import tpu_sc as plsc

_B = 16384
_D = 16
_NC = 2
_NS = 16
_NW = _NC * _NS
_BPW = _B // _NW
_CHUNK = 128
_NCH = _BPW // _CHUNK


def _embed_body(user_hbm, item_hbm, uw_hbm, iw_hbm, out_u, out_i,
                idx_u, idx_i, rows_u, rows_i, sem):
    wid = lax.axis_index("s") * _NC + lax.axis_index("c")
    pltpu.sync_copy(user_hbm.at[wid], idx_u)
    pltpu.sync_copy(item_hbm.at[wid], idx_i)
    copies = []
    for j in range(_NCH):
        copies.append(pltpu.async_copy(
            uw_hbm.at[idx_u.at[j]], rows_u.at[pl.ds(j * _CHUNK, _CHUNK)], sem))
        copies.append(pltpu.async_copy(
            iw_hbm.at[idx_i.at[j]], rows_i.at[pl.ds(j * _CHUNK, _CHUNK)], sem))
    for c in copies:
        c.wait()
    base = wid * _BPW
    pltpu.sync_copy(rows_u, out_u.at[pl.ds(base, _BPW)])
    pltpu.sync_copy(rows_i, out_i.at[pl.ds(base, _BPW)])


@jax.jit
def kernel(user, item, embed_user_w, embed_item_w):
    call = functools.partial(
        pl.kernel,
        mesh=plsc.VectorSubcoreMesh(core_axis_name="c", subcore_axis_name="s"),
        compiler_params=pltpu.CompilerParams(use_tc_tiling_on_sc=False),
        out_type=(
            jax.ShapeDtypeStruct((_B, _D), jnp.float32),
            jax.ShapeDtypeStruct((_B, _D), jnp.float32),
        ),
        scratch_types=[
            pltpu.VMEM((_NCH, _CHUNK), jnp.int32),
            pltpu.VMEM((_NCH, _CHUNK), jnp.int32),
            pltpu.VMEM((_BPW, _D), jnp.float32),
            pltpu.VMEM((_BPW, _D), jnp.float32),
            pltpu.SemaphoreType.DMA,
        ],
    )(_embed_body)
    u2 = user.reshape(_NW, _NCH, _CHUNK)
    i2 = item.reshape(_NW, _NCH, _CHUNK)
    return call(u2, i2, embed_user_w, embed_item_w)
